# baseline (speedup 1.0000x reference)
import functools

import jax
import jax.numpy as jnp
from jax import lax
from jax.experimental import pallas as pl
from jax.experimental.pallas import tpu as pltpu
from jax.experimental.pallas import tpu_sc as plsc

B = 16384
F = 3
VOCAB = 1000000
EMB = 16
R = F * B
NBLK = R // 16

_info = plsc.get_sparse_core_info()
NC = _info.num_cores
NS = _info.num_subcores
L = _info.num_lanes
NW = NC * NS
RPW = R // NW
BPW = NBLK // NW
CHUNK = 128
NCH = RPW // CHUNK

_mesh = plsc.VectorSubcoreMesh(core_axis_name="c", subcore_axis_name="s")


@functools.partial(
    pl.kernel,
    mesh=_mesh,
    compiler_params=pltpu.CompilerParams(use_tc_tiling_on_sc=False),
    out_type=(
        jax.ShapeDtypeStruct((R,), jnp.float32),
        jax.ShapeDtypeStruct((R,), jnp.float32),
        jax.ShapeDtypeStruct((R,), jnp.float32),
    ),
    scratch_types=[
        pltpu.VMEM((NCH, CHUNK), jnp.int32),
        pltpu.VMEM((RPW, EMB), jnp.float32),
        pltpu.VMEM((RPW,), jnp.float32),
        pltpu.VMEM((RPW,), jnp.float32),
        pltpu.VMEM((L,), jnp.float32),
        pltpu.VMEM((L,), jnp.float32),
        pltpu.VMEM((RPW,), jnp.float32),
        pltpu.VMEM((RPW,), jnp.float32),
        pltpu.SemaphoreType.DMA,
    ],
)
def _sc_gather_reduce(so_hbm, fo_hbm, idx_hbm, xv_hbm, w_hbm, p_hbm,
                      s_out, v_out, fo_out,
                      idx_v, rows_v, fo_v, xv_v, w_v, p_v, s_loc, v_loc, sem):
    wid = lax.axis_index("s") * NC + lax.axis_index("c")
    base = wid * RPW

    pltpu.sync_copy(idx_hbm.at[wid], idx_v)
    pltpu.sync_copy(xv_hbm.at[pl.ds(base, RPW)], xv_v)
    pltpu.sync_copy(w_hbm, w_v)
    pltpu.sync_copy(p_hbm, p_v)

    copies = []
    for k in range(NCH):
        copies.append(pltpu.async_copy(
            so_hbm.at[idx_v.at[k]], rows_v.at[pl.ds(k * CHUNK, CHUNK), :], sem))
        copies.append(pltpu.async_copy(
            fo_hbm.at[idx_v.at[k]], fo_v.at[pl.ds(k * CHUNK, CHUNK)], sem))
    for c in copies:
        c.wait()

    w_vec = w_v[...]
    p_vec = p_v[...]

    def block_body(blk, carry):
        del carry
        rbase = blk * L
        xv = xv_v[pl.ds(rbase, L)]
        xv2 = xv * xv
        u = w_vec * xv2
        up = p_vec * xv2
        s_acc = jnp.zeros((L,), jnp.float32)
        v_acc = jnp.zeros((L,), jnp.float32)
        for j in range(L):
            g = rows_v[rbase + j]
            gsq = g * g
            cj = jnp.full((L,), j, dtype=jnp.int32)
            bs = u.at[cj].get(mode="promise_in_bounds")
            bv = up.at[cj].get(mode="promise_in_bounds")
            s_acc = s_acc + bs * gsq
            v_acc = v_acc + bv * gsq
        s_loc[pl.ds(rbase, L)] = s_acc
        v_loc[pl.ds(rbase, L)] = v_acc
        return 0

    lax.fori_loop(0, BPW, block_body, 0)

    pltpu.sync_copy(s_loc, s_out.at[pl.ds(base, RPW)])
    pltpu.sync_copy(v_loc, v_out.at[pl.ds(base, RPW)])
    pltpu.sync_copy(fo_v, fo_out.at[pl.ds(base, RPW)])


def _epilogue_body(s_ref, v_ref, fo_ref, xv_ref, bias_ref, o_ref):
    s0, s1, s2 = s_ref[0], s_ref[1], s_ref[2]
    m = jnp.maximum(s0, jnp.maximum(s1, s2))
    e0 = jnp.exp(s0 - m)
    e1 = jnp.exp(s1 - m)
    e2 = jnp.exp(s2 - m)
    att = (v_ref[0] * e0 + v_ref[1] * e1 + v_ref[2] * e2) / (e0 + e1 + e2)
    first = (fo_ref[0] * xv_ref[0] + fo_ref[1] * xv_ref[1]
             + fo_ref[2] * xv_ref[2])
    o_ref[...] = bias_ref[0, 0] + first + att


def kernel(Xi, Xv, fo_tables, so_tables, W_att, b_att, H, P, bias):
    del b_att
    w = (W_att @ H).astype(jnp.float32)

    idx = (Xi.astype(jnp.int32).T
           + (jnp.arange(F, dtype=jnp.int32) * VOCAB)[:, None]).reshape(NW, NCH, CHUNK)
    xv_flat = Xv.T.reshape(R)
    so_flat = so_tables.reshape(F * VOCAB, EMB)
    fo_flat = fo_tables.reshape(F * VOCAB)

    s_bf, v_bf, fo_g = _sc_gather_reduce(so_flat, fo_flat, idx, xv_flat, w, P)

    s3 = s_bf.reshape(NBLK, L).T.reshape(B, F).T
    v3 = v_bf.reshape(NBLK, L).T.reshape(B, F).T
    fo3 = fo_g.reshape(F, B)
    xv3 = xv_flat.reshape(F, B)

    total = pl.pallas_call(
        _epilogue_body,
        out_shape=jax.ShapeDtypeStruct((B,), jnp.float32),
        in_specs=[
            pl.BlockSpec((F, B), lambda: (0, 0)),
            pl.BlockSpec((F, B), lambda: (0, 0)),
            pl.BlockSpec((F, B), lambda: (0, 0)),
            pl.BlockSpec((F, B), lambda: (0, 0)),
            pl.BlockSpec(memory_space=pltpu.SMEM),
        ],
        out_specs=pl.BlockSpec((B,), lambda: (0,)),
    )(s3, v3, fo3, xv3, jnp.reshape(bias, (1, 1)))
    return total
